# baseline (speedup 1.0000x reference)
import functools

import jax
import jax.numpy as jnp
from jax import lax
from jax.experimental import pallas as pl
from jax.experimental.pallas import tpu as pltpu
from jax.experimental.pallas import tpu_sc as plsc

N_REGIONS = 100000
N_PAIRS = 1000000
N_COMP = 128
B = 16384

NC = 2
NS = 16
NW = NC * NS
B_PER_W = B // NW
CHUNK = 128
N_CHUNKS = B_PER_W // CHUNK
L = 16


@functools.partial(
    pl.kernel,
    mesh=plsc.VectorSubcoreMesh(core_axis_name="c", subcore_axis_name="s"),
    out_type=(
        jax.ShapeDtypeStruct((B, N_COMP), jnp.float32),
        jax.ShapeDtypeStruct((B, N_COMP), jnp.float32),
    ),
    scratch_types=[
        pltpu.VMEM((B_PER_W,), jnp.int32),
        pltpu.VMEM((B_PER_W,), jnp.int32),
        pltpu.VMEM((B_PER_W,), jnp.int32),
        pltpu.VMEM((B_PER_W,), jnp.int32),
        pltpu.VMEM((B_PER_W,), jnp.int32),
        pltpu.VMEM((CHUNK, N_COMP), jnp.float32),
        pltpu.VMEM((CHUNK, N_COMP), jnp.float32),
        pltpu.SemaphoreType.DMA,
        pltpu.SemaphoreType.DMA,
    ],
)
def _pca_gather(idx_hbm, pair_flat_hbm, proj_hbm, out_i_hbm, out_j_hbm,
                idx_v, ei_v, ej_v, ri_v, rj_v, buf0, buf1, gsem, wsem):
    wid = lax.axis_index("s") * NC + lax.axis_index("c")
    base = wid * B_PER_W

    pltpu.sync_copy(idx_hbm.at[pl.ds(base, B_PER_W)], idx_v)

    def offs_body(k, carry):
        v = idx_v[pl.ds(k * L, L)]
        two = v + v
        ei_v[pl.ds(k * L, L)] = two
        ej_v[pl.ds(k * L, L)] = two + 1
        return carry

    lax.fori_loop(0, B_PER_W // L, offs_body, 0)

    for src, dst in ((ei_v, ri_v), (ej_v, rj_v)):
        for c in range(N_CHUNKS):
            pltpu.async_copy(
                pair_flat_hbm.at[src.at[pl.ds(c * CHUNK, CHUNK)]],
                dst.at[pl.ds(c * CHUNK, CHUNK)],
                gsem,
            ).wait()

    for rv, out_hbm in ((ri_v, out_i_hbm), (rj_v, out_j_hbm)):
        for c in range(N_CHUNKS):
            buf = buf0 if c % 2 == 0 else buf1
            pltpu.async_copy(
                proj_hbm.at[rv.at[pl.ds(c * CHUNK, CHUNK)]],
                buf,
                gsem,
            ).wait()
            pltpu.sync_copy(buf, out_hbm.at[pl.ds(base + c * CHUNK, CHUNK)])


def kernel(x, expanded_idx, pair_table, cached_projection):
    del x
    return _pca_gather(expanded_idx, pair_table.reshape(-1), cached_projection)


# speedup vs baseline: 15.4096x; 15.4096x over previous
import functools

import jax
import jax.numpy as jnp
from jax import lax
from jax.experimental import pallas as pl
from jax.experimental.pallas import tpu as pltpu
from jax.experimental.pallas import tpu_sc as plsc

N_REGIONS = 100000
N_PAIRS = 1000000
N_COMP = 128
B = 16384

NC = 2
NS = 16
NW = NC * NS
B_PER_W = B // NW
CHUNK = 128
N_CHUNKS = B_PER_W // CHUNK
L = 16


@functools.partial(
    pl.kernel,
    mesh=plsc.VectorSubcoreMesh(core_axis_name="c", subcore_axis_name="s"),
    out_type=(
        jax.ShapeDtypeStruct((B, N_COMP), jnp.float32),
        jax.ShapeDtypeStruct((B, N_COMP), jnp.float32),
    ),
    scratch_types=[
        pltpu.VMEM((B_PER_W,), jnp.int32),
        pltpu.VMEM((B_PER_W,), jnp.int32),
        pltpu.VMEM((B_PER_W,), jnp.int32),
        pltpu.VMEM((CHUNK, N_COMP), jnp.float32),
        pltpu.VMEM((CHUNK, N_COMP), jnp.float32),
        pltpu.SemaphoreType.DMA,
        pltpu.SemaphoreType.DMA,
    ],
)
def _pca_gather(idx_hbm, pi_hbm, pj_hbm, proj_hbm, out_i_hbm, out_j_hbm,
                idx_v, ri_v, rj_v, buf0, buf1, gsem, wsem):
    wid = lax.axis_index("s") * NC + lax.axis_index("c")
    base = wid * B_PER_W

    pltpu.sync_copy(idx_hbm.at[pl.ds(base, B_PER_W)], idx_v)

    for src_hbm, dst in ((pi_hbm, ri_v), (pj_hbm, rj_v)):
        for c in range(N_CHUNKS):
            pltpu.async_copy(
                src_hbm.at[idx_v.at[pl.ds(c * CHUNK, CHUNK)]],
                dst.at[pl.ds(c * CHUNK, CHUNK)],
                gsem,
            ).wait()

    for rv, out_hbm in ((ri_v, out_i_hbm), (rj_v, out_j_hbm)):
        for c in range(N_CHUNKS):
            buf = buf0 if c % 2 == 0 else buf1
            pltpu.async_copy(
                proj_hbm.at[rv.at[pl.ds(c * CHUNK, CHUNK)]],
                buf,
                gsem,
            ).wait()
            pltpu.sync_copy(buf, out_hbm.at[pl.ds(base + c * CHUNK, CHUNK)])


def kernel(x, expanded_idx, pair_table, cached_projection):
    del x
    pair_i = jnp.minimum(pair_table[:, 0], N_REGIONS - 1)
    pair_j = jnp.minimum(pair_table[:, 1], N_REGIONS - 1)
    return _pca_gather(expanded_idx, pair_i, pair_j, cached_projection)
